# Initial kernel scaffold; baseline (speedup 1.0000x reference)
import jax
import jax.numpy as jnp
from jax.experimental import pallas as pl


def kernel(feats, coords, weight, bias):
    raise NotImplementedError("write your pallas kernel here")


# speedup vs baseline: 1.3156x; 1.3156x over previous
import dataclasses
import functools

import jax
import jax.numpy as jnp
from jax import lax
from jax.experimental import pallas as pl
from jax.experimental.pallas import tpu as pltpu
from jax.experimental.pallas import tpu_sc as plsc

D = H = W = 128
B = 4
N = 100000
Ci = Co = 32
TAB = B * D * H * W
TABP = TAB + 8
NW = 32
CH = 128
NP = 102400
NPV = NP + 8
PT_CH = NP // (NW * CH)
OFFS = [(dz, dy, dx)
        for dz in (-1, 0, 1) for dy in (-1, 0, 1) for dx in (-1, 0, 1)
        if (dz, dy, dx) != (0, 0, 0)]
KN = len(OFFS)
RTOT = KN * NP
RW = RTOT // NW
WK_CH = RW // CH
MBLK = 1024


def _sc_mesh():
    return plsc.VectorSubcoreMesh(core_axis_name="c", subcore_axis_name="s")


def _sc_params():
    cp = pltpu.CompilerParams()
    fields = pltpu.CompilerParams.__dataclass_fields__
    if "needs_layout_passes" in fields:
        cp = dataclasses.replace(cp, needs_layout_passes=False)
    if "use_tc_tiling_on_sc" in fields:
        cp = dataclasses.replace(cp, use_tc_tiling_on_sc=False)
    return cp


def _wid():
    return lax.axis_index("c") * 16 + lax.axis_index("s")


def _scatter_body(lin_hbm, ids_hbm, table_hbm, idx_v, val_v):
    wid = _wid()

    @pl.loop(0, PT_CH)
    def _(c):
        row = wid * PT_CH + c
        pltpu.sync_copy(lin_hbm.at[row], idx_v)
        pltpu.sync_copy(ids_hbm.at[row], val_v)
        pltpu.sync_copy(val_v, table_hbm.at[idx_v])


def _scatter_table(lin_pad, ids_pad):
    k = pl.kernel(
        _scatter_body,
        out_type=jax.ShapeDtypeStruct((TABP,), jnp.int32),
        mesh=_sc_mesh(),
        scratch_types=[
            pltpu.VMEM((CH,), jnp.int32),
            pltpu.VMEM((CH,), jnp.int32),
        ],
    )
    return k(lin_pad.reshape(NW * PT_CH, CH), ids_pad.reshape(NW * PT_CH, CH))


def _qidx_body(lin_ref, offs_ref, q_ref):
    k = pl.program_id(0)
    l = lin_ref[...]
    x = l & 127
    y = (l >> 7) & 127
    z = (l >> 14) & 127
    dz = offs_ref[0, k]
    dy = offs_ref[1, k]
    dx = offs_ref[2, k]
    xn = x + dx
    yn = y + dy
    zn = z + dz
    ok = ((xn >= 0) & (xn <= 127) & (yn >= 0) & (yn <= 127)
          & (zn >= 0) & (zn <= 127))
    q = l + (dz * (H * W) + dy * W + dx)
    qm = jnp.where(ok, q, TAB)
    q_ref[...] = jnp.minimum(jnp.maximum(qm, 0), TAB)[None]


def _make_qidx(lin_pad, offs):
    rows = NP // 128
    grid = (KN, rows // 8)
    return pl.pallas_call(
        _qidx_body,
        grid=grid,
        in_specs=[
            pl.BlockSpec((8, 128), lambda k, r: (r, 0)),
            pl.BlockSpec(memory_space=pltpu.SMEM),
        ],
        out_specs=pl.BlockSpec((1, 8, 128), lambda k, r: (k, r, 0)),
        out_shape=jax.ShapeDtypeStruct((KN, rows, 128), jnp.int32),
    )(lin_pad.reshape(rows, 128), offs)


def _lookup_body(qidx_hbm, table_hbm, keys_hbm, feats_hbm, g_hbm,
                 keys_v, q_v, nid_v, fidx_v, fbuf_v):
    wid = _wid()
    pltpu.sync_copy(keys_hbm, keys_v)

    @pl.loop(0, WK_CH)
    def _(c):
        pltpu.sync_copy(qidx_hbm.at[wid * WK_CH + c], q_v)
        pltpu.sync_copy(table_hbm.at[q_v], nid_v)
        for s in range(CH // 16):
            sl = pl.ds(s * 16, 16)
            q16 = q_v[sl]
            nraw = nid_v[sl]
            nc = jnp.minimum(jnp.maximum(nraw, 0), NPV - 1)
            key = plsc.load_gather(keys_v, [nc])
            fidx_v[sl] = jnp.where(key == q16, nc, NP)
        pltpu.sync_copy(feats_hbm.at[fidx_v], fbuf_v)
        pltpu.sync_copy(fbuf_v, g_hbm.at[pl.ds(wid * RW + c * CH, CH)])


def _gather_feats(qidx, table, keys, feats_pad):
    k = pl.kernel(
        _lookup_body,
        out_type=jax.ShapeDtypeStruct((RTOT, Ci), jnp.float32),
        mesh=_sc_mesh(),
        scratch_types=[
            pltpu.VMEM((NPV,), jnp.int32),
            pltpu.VMEM((CH,), jnp.int32),
            pltpu.VMEM((CH,), jnp.int32),
            pltpu.VMEM((CH,), jnp.int32),
            pltpu.VMEM((CH, Ci), jnp.float32),
        ],
        compiler_params=_sc_params(),
    )
    return k(qidx.reshape(NW * WK_CH, CH), table, keys, feats_pad)


def _mm_body(g_ref, f_ref, w_ref, b_ref, o_ref):
    k = pl.program_id(1)

    @pl.when(k == 0)
    def _():
        o_ref[...] = jnp.broadcast_to(b_ref[...], (MBLK, Co))

    @pl.when(k < KN)
    def _():
        o_ref[...] += jnp.dot(g_ref[0], w_ref[0],
                              preferred_element_type=jnp.float32)

    @pl.when(k == KN)
    def _():
        o_ref[...] += jnp.dot(f_ref[...], w_ref[0],
                              preferred_element_type=jnp.float32)


def _matmul(g, feats_pad, wn, bias):
    grid = (NP // MBLK, KN + 1)
    return pl.pallas_call(
        _mm_body,
        grid=grid,
        in_specs=[
            pl.BlockSpec((1, MBLK, Ci),
                         lambda r, k: (jnp.minimum(k, KN - 1), r, 0)),
            pl.BlockSpec((MBLK, Ci), lambda r, k: (r, 0)),
            pl.BlockSpec((1, Ci, Co), lambda r, k: (k, 0, 0)),
            pl.BlockSpec((1, Co), lambda r, k: (0, 0)),
        ],
        out_specs=pl.BlockSpec((MBLK, Co), lambda r, k: (r, 0)),
        out_shape=jax.ShapeDtypeStruct((NP, Co), jnp.float32),
    )(g.reshape(KN, NP, Ci), feats_pad[:NP], wn, bias.reshape(1, Co))


def kernel(feats, coords, weight, bias):
    coords = coords.astype(jnp.int32)
    lin = ((coords[:, 0] * D + coords[:, 1]) * H + coords[:, 2]) * W \
        + coords[:, 3]
    lin_pad = jnp.full((NP,), TAB, jnp.int32).at[:N].set(lin)
    keys = jnp.full((NPV,), -1, jnp.int32).at[:N].set(lin)
    ids_pad = jnp.concatenate(
        [jnp.arange(N, dtype=jnp.int32),
         jnp.full((NP - N,), N, jnp.int32)])
    feats_pad = jnp.zeros((NPV, Ci), jnp.float32).at[:N].set(feats)
    offs = jnp.array([[o[i] for o in OFFS] for i in range(3)], jnp.int32)
    wt = jnp.transpose(weight, (1, 2, 3, 4, 0)).reshape(27, Ci, Co)
    kidx = [(dz + 1) * 9 + (dy + 1) * 3 + (dx + 1) for dz, dy, dx in OFFS]
    wn = jnp.concatenate([wt[jnp.array(kidx)], wt[13:14]], axis=0)

    table = _scatter_table(lin_pad, ids_pad)
    qidx = _make_qidx(lin_pad, offs)
    g = _gather_feats(qidx, table, keys, feats_pad)
    out = _matmul(g, feats_pad, wn, bias)
    return out[:N]


# speedup vs baseline: 2.3715x; 1.8026x over previous
import dataclasses

import jax
import jax.numpy as jnp
from jax import lax
from jax.experimental import pallas as pl
from jax.experimental.pallas import tpu as pltpu
from jax.experimental.pallas import tpu_sc as plsc

D = H = W = 128
B = 4
N = 100000
Ci = Co = 32
TAB = B * D * H * W
TABP = TAB + 8
NW = 32
CH = 128
NP = 102400
NPV = NP + 8
KEYR = N + 8
PPT = NP // NW
PC = PPT // CH
OFFS = [(dz, dy, dx)
        for dz in (-1, 0, 1) for dy in (-1, 0, 1) for dx in (-1, 0, 1)
        if (dz, dy, dx) != (0, 0, 0)]
KN = len(OFFS)
QC = KN * CH
RTOT = KN * NP
RW = RTOT // NW
CBJ = 25
NCB = RW // (CBJ * CH)
MBLK = 1024


def _sc_mesh():
    return plsc.VectorSubcoreMesh(core_axis_name="c", subcore_axis_name="s")


def _sc_params():
    cp = pltpu.CompilerParams()
    fields = pltpu.CompilerParams.__dataclass_fields__
    if "needs_layout_passes" in fields:
        cp = dataclasses.replace(cp, needs_layout_passes=False)
    if "use_tc_tiling_on_sc" in fields:
        cp = dataclasses.replace(cp, use_tc_tiling_on_sc=False)
    return cp


def _wid():
    return lax.axis_index("c") * 16 + lax.axis_index("s")


def _scatter_body(lin_hbm, ids_hbm, table_hbm, lin_v, ids_v, sem):
    wid = _wid()
    pltpu.sync_copy(lin_hbm.at[wid], lin_v)
    pltpu.sync_copy(ids_hbm.at[wid], ids_v)

    @pl.loop(0, PC)
    def _(c):
        pltpu.async_copy(ids_v.at[c], table_hbm.at[lin_v.at[c]], sem)

    @pl.loop(0, PC)
    def _(c):
        pltpu.make_async_copy(ids_v.at[c], table_hbm.at[lin_v.at[c]],
                              sem).wait()


def _scatter_table(lin_pad, ids_pad):
    k = pl.kernel(
        _scatter_body,
        out_type=jax.ShapeDtypeStruct((TABP,), jnp.int32),
        mesh=_sc_mesh(),
        scratch_types=[
            pltpu.VMEM((PC, CH), jnp.int32),
            pltpu.VMEM((PC, CH), jnp.int32),
            pltpu.SemaphoreType.DMA,
        ],
        compiler_params=_sc_params(),
    )
    return k(lin_pad.reshape(NW, PC, CH), ids_pad.reshape(NW, PC, CH))


def _mk_masks(l):
    x = l & 127
    y = (l >> 7) & 127
    z = (l >> 14) & 127
    vp = l < TAB
    return {
        (0, -1): (x > 0) & vp, (0, 1): (x < 127) & vp,
        (1, -1): (y > 0) & vp, (1, 1): (y < 127) & vp,
        (2, -1): (z > 0) & vp, (2, 1): (z < 127) & vp,
        None: vp,
    }


def _compute_q(lin_v, q_v, b, c):
    for s in range(CH // 16):
        l = lin_v[pl.ds(c * CH + s * 16, 16)]
        m = _mk_masks(l)
        for ki, (dz, dy, dx) in enumerate(OFFS):
            ok = m[None]
            for d, dd in ((2, dz), (1, dy), (0, dx)):
                if dd != 0:
                    ok = ok & m[(d, dd)]
            off = dz * (H * W) + dy * W + dx
            q_v[b, ki, pl.ds(s * 16, 16)] = jnp.where(ok, l + off, TAB)


def _lookup_body(lin_hbm, keys_hbm, table_hbm, fidx_hbm,
                 keys_v, lin_v, q_v, nid_v, f_v,
                 sem_g0, sem_g1, sem_w0, sem_w1):
    wid = _wid()
    pltpu.sync_copy(keys_hbm, keys_v)
    pltpu.sync_copy(lin_hbm.at[wid], lin_v)
    sem_gs = (sem_g0, sem_g1)

    def fire_gather(b):
        @pl.loop(0, KN)
        def _(ki):
            pltpu.async_copy(table_hbm.at[q_v.at[b, ki]], nid_v.at[b, ki],
                             sem_gs[b])

    def wait_gather(b):
        @pl.loop(0, KN)
        def _(ki):
            pltpu.make_async_copy(table_hbm.at[q_v.at[b, ki]],
                                  nid_v.at[b, ki], sem_gs[b]).wait()

    def verify_store(c, b, sem_w, drain):
        @pl.when(drain)
        def _():
            pltpu.make_async_copy(f_v.at[b], fidx_hbm.at[0], sem_w).wait()
        for ki in range(KN):
            for s in range(CH // 16):
                sl = pl.ds(s * 16, 16)
                nc = jnp.minimum(jnp.maximum(nid_v[b, ki, sl], 0), KEYR - 1)
                key = plsc.load_gather(keys_v, [nc])
                f_v[b, ki, sl] = jnp.where(key == q_v[b, ki, sl], nc, NP)
        pltpu.async_copy(f_v.at[b], fidx_hbm.at[wid * PC + c], sem_w)

    _compute_q(lin_v, q_v, 0, 0)
    fire_gather(0)

    @pl.loop(0, PC // 2)
    def _(i):
        c0 = i * 2
        _compute_q(lin_v, q_v, 1, c0 + 1)
        fire_gather(1)
        wait_gather(0)
        verify_store(c0, 0, sem_w0, i > 0)

        @pl.when(c0 + 2 < PC)
        def _():
            _compute_q(lin_v, q_v, 0, c0 + 2)
            fire_gather(0)
        wait_gather(1)
        verify_store(c0 + 1, 1, sem_w1, i > 0)

    wait_gather(0)
    verify_store(PC - 1, 0, sem_w0, True)
    pltpu.make_async_copy(f_v.at[0], fidx_hbm.at[0], sem_w0).wait()
    pltpu.make_async_copy(f_v.at[1], fidx_hbm.at[0], sem_w1).wait()


def _lookup(lin_pad, keys, table):
    k = pl.kernel(
        _lookup_body,
        out_type=jax.ShapeDtypeStruct((NW * PC, KN, CH), jnp.int32),
        mesh=_sc_mesh(),
        scratch_types=[
            pltpu.VMEM((KEYR,), jnp.int32),
            pltpu.VMEM((PPT,), jnp.int32),
            pltpu.VMEM((2, KN, CH), jnp.int32),
            pltpu.VMEM((2, KN, CH), jnp.int32),
            pltpu.VMEM((2, KN, CH), jnp.int32),
            pltpu.SemaphoreType.DMA,
            pltpu.SemaphoreType.DMA,
            pltpu.SemaphoreType.DMA,
            pltpu.SemaphoreType.DMA,
        ],
        compiler_params=_sc_params(),
    )
    return k(lin_pad.reshape(NW, PPT), keys, table)


def _featgather_body(fidx_hbm, fpk_hbm, g_hbm,
                     idx_v, fbuf_v, sem_f0, sem_f1, sem_w0, sem_w1):
    wid = _wid()
    base = wid * (RW // CH)
    sem_fs = (sem_f0, sem_f1)
    sem_ws = (sem_w0, sem_w1)

    def load_idx(c, p):
        pltpu.sync_copy(fidx_hbm.at[pl.ds(base + c * CBJ, CBJ)],
                        idx_v.at[p])

    def fire_gather(p):
        @pl.loop(0, CBJ)
        def _(j):
            pltpu.async_copy(fpk_hbm.at[idx_v.at[p, j]], fbuf_v.at[p, j],
                             sem_fs[p])

    def wait_gather(p):
        @pl.loop(0, CBJ)
        def _(j):
            pltpu.make_async_copy(fpk_hbm.at[idx_v.at[p, j]],
                                  fbuf_v.at[p, j], sem_fs[p]).wait()

    def fire_write(c, p):
        pltpu.async_copy(fbuf_v.at[p],
                         g_hbm.at[pl.ds(base + c * CBJ, CBJ)], sem_ws[p])

    def wait_write(p):
        pltpu.make_async_copy(fbuf_v.at[p], g_hbm.at[pl.ds(base, CBJ)],
                              sem_ws[p]).wait()

    load_idx(0, 0)
    fire_gather(0)

    @pl.loop(0, NCB // 2)
    def _(i):
        c0 = i * 2

        @pl.when(i > 0)
        def _():
            wait_write(1)
        load_idx(c0 + 1, 1)
        fire_gather(1)
        wait_gather(0)
        fire_write(c0, 0)

        @pl.when(c0 + 2 < NCB)
        def _():
            wait_write(0)
            load_idx(c0 + 2, 0)
            fire_gather(0)
        wait_gather(1)
        fire_write(c0 + 1, 1)

    wait_write(0)
    wait_write(1)


def _featgather(fidx, fpk):
    k = pl.kernel(
        _featgather_body,
        out_type=jax.ShapeDtypeStruct((RTOT // CH, CH, Ci // 2), jnp.int32),
        mesh=_sc_mesh(),
        scratch_types=[
            pltpu.VMEM((2, CBJ, CH), jnp.int32),
            pltpu.VMEM((2, CBJ, CH, Ci // 2), jnp.int32),
            pltpu.SemaphoreType.DMA,
            pltpu.SemaphoreType.DMA,
            pltpu.SemaphoreType.DMA,
            pltpu.SemaphoreType.DMA,
        ],
        compiler_params=_sc_params(),
    )
    return k(fidx.reshape(RTOT // CH, CH), fpk)


def _mm_body(g_ref, f_ref, wlo_ref, whi_ref, wc_ref, b_ref, o_ref):
    k = pl.program_id(1)

    @pl.when(k == 0)
    def _():
        o_ref[...] = jnp.broadcast_to(b_ref[...], (MBLK, Co))

    @pl.when(k < KN)
    def _():
        gi = g_ref[...].reshape(MBLK, Ci // 2)
        lo = jax.lax.bitcast_convert_type(gi << 16, jnp.float32)
        hi = jax.lax.bitcast_convert_type(
            gi & jnp.int32(-65536), jnp.float32)
        o_ref[...] += (jnp.dot(lo, wlo_ref[0],
                               preferred_element_type=jnp.float32)
                       + jnp.dot(hi, whi_ref[0],
                                 preferred_element_type=jnp.float32))

    @pl.when(k == KN)
    def _():
        o_ref[...] += jnp.dot(f_ref[...], wc_ref[0],
                              preferred_element_type=jnp.float32)


def _matmul(g, feats_pad, wlo, whi, wc, bias):
    npc = NP // CH
    bpc = MBLK // CH
    g4 = g.reshape(npc, KN, CH, Ci // 2)
    grid = (npc // bpc, KN + 1)
    wsel = lambda r, k: (jnp.minimum(k, KN - 1), 0, 0)
    return pl.pallas_call(
        _mm_body,
        grid=grid,
        in_specs=[
            pl.BlockSpec((bpc, 1, CH, Ci // 2),
                         lambda r, k: (r, jnp.minimum(k, KN - 1), 0, 0)),
            pl.BlockSpec((MBLK, Ci), lambda r, k: (r, 0)),
            pl.BlockSpec((1, Ci // 2, Co), wsel),
            pl.BlockSpec((1, Ci // 2, Co), wsel),
            pl.BlockSpec((1, Ci, Co), lambda r, k: (0, 0, 0)),
            pl.BlockSpec((1, Co), lambda r, k: (0, 0)),
        ],
        out_specs=pl.BlockSpec((MBLK, Co), lambda r, k: (r, 0)),
        out_shape=jax.ShapeDtypeStruct((NP, Co), jnp.float32),
    )(g4, feats_pad[:NP], wlo, whi, wc, bias.reshape(1, Co))


def kernel(feats, coords, weight, bias):
    coords = coords.astype(jnp.int32)
    lin = ((coords[:, 0] * D + coords[:, 1]) * H + coords[:, 2]) * W \
        + coords[:, 3]
    lin_pad = jnp.full((NP,), TAB, jnp.int32).at[:N].set(lin)
    keys = jnp.full((KEYR,), -1, jnp.int32).at[:N].set(lin)
    ids_pad = jnp.concatenate(
        [jnp.arange(N, dtype=jnp.int32),
         jnp.full((NP - N,), N, jnp.int32)])
    feats_pad = jnp.zeros((NPV, Ci), jnp.float32).at[:N].set(feats)
    fpk = jax.lax.bitcast_convert_type(
        feats_pad.astype(jnp.bfloat16).reshape(NPV, Ci // 2, 2), jnp.int32)
    wt = jnp.transpose(weight, (1, 2, 3, 4, 0)).reshape(27, Ci, Co)
    kidx = [(dz + 1) * 9 + (dy + 1) * 3 + (dx + 1) for dz, dy, dx in OFFS]
    wn = wt[jnp.array(kidx)]
    wbf = wn.astype(jnp.bfloat16).astype(jnp.float32)
    wlo = wbf[:, 0::2, :]
    whi = wbf[:, 1::2, :]
    wc = wt[13:14]

    table = _scatter_table(lin_pad, ids_pad)
    fidx = _lookup(lin_pad, keys, table)
    g = _featgather(fidx, fpk)
    out = _matmul(g, feats_pad, wlo, whi, wc, bias)
    return out[:N]


# speedup vs baseline: 9.3261x; 3.9326x over previous
import dataclasses

import jax
import jax.numpy as jnp
from jax import lax
from jax.experimental import pallas as pl
from jax.experimental.pallas import tpu as pltpu
from jax.experimental.pallas import tpu_sc as plsc

D = H = W = 128
B = 4
N = 100000
Ci = Co = 32
TAB = B * D * H * W
TABP = TAB + 8
NW = 32
CH = 128
NP = 102400
ZSP = 4096
NPV = NP + ZSP + 8
KEYR = N + 8
PPT = NP // NW
PC = PPT // CH
OFFS = [(dz, dy, dx)
        for dz in (-1, 0, 1) for dy in (-1, 0, 1) for dx in (-1, 0, 1)
        if (dz, dy, dx) != (0, 0, 0)]
KN = len(OFFS)
QC = KN * CH
RTOT = KN * NP
RW = RTOT // NW
CBJ = 25
NCB = RW // (CBJ * CH)
MBLK = 1024


def _sc_mesh():
    return plsc.VectorSubcoreMesh(core_axis_name="c", subcore_axis_name="s")


def _sc_params():
    cp = pltpu.CompilerParams()
    fields = pltpu.CompilerParams.__dataclass_fields__
    if "needs_layout_passes" in fields:
        cp = dataclasses.replace(cp, needs_layout_passes=False)
    if "use_tc_tiling_on_sc" in fields:
        cp = dataclasses.replace(cp, use_tc_tiling_on_sc=False)
    return cp


def _wid():
    return lax.axis_index("c") * 16 + lax.axis_index("s")


def _scatter_body(lin_hbm, ids_hbm, table_hbm, lin_v, ids_v, sem):
    wid = _wid()
    pltpu.sync_copy(lin_hbm.at[wid], lin_v)
    pltpu.sync_copy(ids_hbm.at[wid], ids_v)

    @pl.loop(0, PC)
    def _(c):
        pltpu.async_copy(ids_v.at[c], table_hbm.at[lin_v.at[c]], sem)

    @pl.loop(0, PC)
    def _(c):
        pltpu.make_async_copy(ids_v.at[c], table_hbm.at[lin_v.at[c]],
                              sem).wait()


def _scatter_table(lin_pad, ids_pad):
    k = pl.kernel(
        _scatter_body,
        out_type=jax.ShapeDtypeStruct((TABP,), jnp.int32),
        mesh=_sc_mesh(),
        scratch_types=[
            pltpu.VMEM((PC, CH), jnp.int32),
            pltpu.VMEM((PC, CH), jnp.int32),
            pltpu.SemaphoreType.DMA,
        ],
        compiler_params=_sc_params(),
    )
    return k(lin_pad.reshape(NW, PC, CH), ids_pad.reshape(NW, PC, CH))


def _mk_masks(l):
    x = l & 127
    y = (l >> 7) & 127
    z = (l >> 14) & 127
    vp = l < TAB
    return {
        (0, -1): (x > 0) & vp, (0, 1): (x < 127) & vp,
        (1, -1): (y > 0) & vp, (1, 1): (y < 127) & vp,
        (2, -1): (z > 0) & vp, (2, 1): (z < 127) & vp,
        None: vp,
    }


def _compute_q(lin_v, q_v, b, c):
    for s in range(CH // 16):
        l = lin_v[pl.ds(c * CH + s * 16, 16)]
        m = _mk_masks(l)
        for ki, (dz, dy, dx) in enumerate(OFFS):
            ok = m[None]
            for d, dd in ((2, dz), (1, dy), (0, dx)):
                if dd != 0:
                    ok = ok & m[(d, dd)]
            off = dz * (H * W) + dy * W + dx
            q_v[b, ki, pl.ds(s * 16, 16)] = jnp.where(ok, l + off, TAB)


def _lookup_body(lin_hbm, keys_hbm, table_hbm, fidx_hbm,
                 keys_v, lin_v, q_v, nid_v, f_v,
                 sem_g0, sem_g1, sem_w0, sem_w1):
    wid = _wid()
    pltpu.sync_copy(keys_hbm, keys_v)
    pltpu.sync_copy(lin_hbm.at[wid], lin_v)
    sem_gs = (sem_g0, sem_g1)

    def fire_gather(b):
        @pl.loop(0, KN)
        def _(ki):
            pltpu.async_copy(table_hbm.at[q_v.at[b, ki]], nid_v.at[b, ki],
                             sem_gs[b])

    def wait_gather(b):
        @pl.loop(0, KN)
        def _(ki):
            pltpu.make_async_copy(table_hbm.at[q_v.at[b, ki]],
                                  nid_v.at[b, ki], sem_gs[b]).wait()

    def verify_store(c, b, sem_w, drain):
        @pl.when(drain)
        def _():
            pltpu.make_async_copy(f_v.at[b], fidx_hbm.at[0], sem_w).wait()
        li = jax.lax.iota(jnp.int32, 16)
        for ki in range(KN):
            for s in range(CH // 16):
                sl = pl.ds(s * 16, 16)
                q16 = q_v[b, ki, sl]
                nc = jnp.minimum(jnp.maximum(nid_v[b, ki, sl], 0), KEYR - 1)
                key = plsc.load_gather(keys_v, [nc])
                dummy = NP + ((q16 + li * 257 + ki * 131 + s * 17)
                              & (ZSP - 1))
                f_v[b, ki, sl] = jnp.where(key == q16, nc, dummy)
        pltpu.async_copy(f_v.at[b], fidx_hbm.at[wid * PC + c], sem_w)

    _compute_q(lin_v, q_v, 0, 0)
    fire_gather(0)

    @pl.loop(0, PC // 2)
    def _(i):
        c0 = i * 2
        _compute_q(lin_v, q_v, 1, c0 + 1)
        fire_gather(1)
        wait_gather(0)
        verify_store(c0, 0, sem_w0, i > 0)

        @pl.when(c0 + 2 < PC)
        def _():
            _compute_q(lin_v, q_v, 0, c0 + 2)
            fire_gather(0)
        wait_gather(1)
        verify_store(c0 + 1, 1, sem_w1, i > 0)

    wait_gather(0)
    verify_store(PC - 1, 0, sem_w0, True)
    pltpu.make_async_copy(f_v.at[0], fidx_hbm.at[0], sem_w0).wait()
    pltpu.make_async_copy(f_v.at[1], fidx_hbm.at[0], sem_w1).wait()


def _lookup(lin_pad, keys, table):
    k = pl.kernel(
        _lookup_body,
        out_type=jax.ShapeDtypeStruct((NW * PC, KN, CH), jnp.int32),
        mesh=_sc_mesh(),
        scratch_types=[
            pltpu.VMEM((KEYR,), jnp.int32),
            pltpu.VMEM((PPT,), jnp.int32),
            pltpu.VMEM((2, KN, CH), jnp.int32),
            pltpu.VMEM((2, KN, CH), jnp.int32),
            pltpu.VMEM((2, KN, CH), jnp.int32),
            pltpu.SemaphoreType.DMA,
            pltpu.SemaphoreType.DMA,
            pltpu.SemaphoreType.DMA,
            pltpu.SemaphoreType.DMA,
        ],
        compiler_params=_sc_params(),
    )
    return k(lin_pad.reshape(NW, PPT), keys, table)


def _featgather_body(fidx_hbm, fpk_hbm, g_hbm,
                     idx_v, fbuf_v, sem_f0, sem_f1, sem_w0, sem_w1):
    wid = _wid()
    base = wid * (RW // CH)
    sem_fs = (sem_f0, sem_f1)
    sem_ws = (sem_w0, sem_w1)

    def load_idx(c, p):
        pltpu.sync_copy(fidx_hbm.at[pl.ds(base + c * CBJ, CBJ)],
                        idx_v.at[p])

    def fire_gather(p):
        @pl.loop(0, CBJ)
        def _(j):
            pltpu.async_copy(fpk_hbm.at[idx_v.at[p, j]], fbuf_v.at[p, j],
                             sem_fs[p])

    def wait_gather(p):
        @pl.loop(0, CBJ)
        def _(j):
            pltpu.make_async_copy(fpk_hbm.at[idx_v.at[p, j]],
                                  fbuf_v.at[p, j], sem_fs[p]).wait()

    def fire_write(c, p):
        pltpu.async_copy(fbuf_v.at[p],
                         g_hbm.at[pl.ds(base + c * CBJ, CBJ)], sem_ws[p])

    def wait_write(p):
        pltpu.make_async_copy(fbuf_v.at[p], g_hbm.at[pl.ds(base, CBJ)],
                              sem_ws[p]).wait()

    load_idx(0, 0)
    fire_gather(0)

    @pl.loop(0, NCB // 2)
    def _(i):
        c0 = i * 2

        @pl.when(i > 0)
        def _():
            wait_write(1)
        load_idx(c0 + 1, 1)
        fire_gather(1)
        wait_gather(0)
        fire_write(c0, 0)

        @pl.when(c0 + 2 < NCB)
        def _():
            wait_write(0)
            load_idx(c0 + 2, 0)
            fire_gather(0)
        wait_gather(1)
        fire_write(c0 + 1, 1)

    wait_write(0)
    wait_write(1)


def _featgather(fidx, fpk):
    k = pl.kernel(
        _featgather_body,
        out_type=jax.ShapeDtypeStruct((RTOT // CH, CH, Ci // 2), jnp.int32),
        mesh=_sc_mesh(),
        scratch_types=[
            pltpu.VMEM((2, CBJ, CH), jnp.int32),
            pltpu.VMEM((2, CBJ, CH, Ci // 2), jnp.int32),
            pltpu.SemaphoreType.DMA,
            pltpu.SemaphoreType.DMA,
            pltpu.SemaphoreType.DMA,
            pltpu.SemaphoreType.DMA,
        ],
        compiler_params=_sc_params(),
    )
    return k(fidx.reshape(RTOT // CH, CH), fpk)


def _mm_body(g_ref, f_ref, wlo_ref, whi_ref, wc_ref, b_ref, o_ref):
    k = pl.program_id(1)

    @pl.when(k == 0)
    def _():
        o_ref[...] = jnp.broadcast_to(b_ref[...], (MBLK, Co))

    @pl.when(k < KN)
    def _():
        gi = g_ref[...].reshape(MBLK, Ci // 2)
        lo = jax.lax.bitcast_convert_type(gi << 16, jnp.float32)
        hi = jax.lax.bitcast_convert_type(
            gi & jnp.int32(-65536), jnp.float32)
        o_ref[...] += (jnp.dot(lo, wlo_ref[0],
                               preferred_element_type=jnp.float32)
                       + jnp.dot(hi, whi_ref[0],
                                 preferred_element_type=jnp.float32))

    @pl.when(k == KN)
    def _():
        o_ref[...] += jnp.dot(f_ref[...], wc_ref[0],
                              preferred_element_type=jnp.float32)


def _matmul(g, feats_pad, wlo, whi, wc, bias):
    npc = NP // CH
    bpc = MBLK // CH
    g4 = g.reshape(npc, KN, CH, Ci // 2)
    grid = (npc // bpc, KN + 1)
    wsel = lambda r, k: (jnp.minimum(k, KN - 1), 0, 0)
    return pl.pallas_call(
        _mm_body,
        grid=grid,
        in_specs=[
            pl.BlockSpec((bpc, 1, CH, Ci // 2),
                         lambda r, k: (r, jnp.minimum(k, KN - 1), 0, 0)),
            pl.BlockSpec((MBLK, Ci), lambda r, k: (r, 0)),
            pl.BlockSpec((1, Ci // 2, Co), wsel),
            pl.BlockSpec((1, Ci // 2, Co), wsel),
            pl.BlockSpec((1, Ci, Co), lambda r, k: (0, 0, 0)),
            pl.BlockSpec((1, Co), lambda r, k: (0, 0)),
        ],
        out_specs=pl.BlockSpec((MBLK, Co), lambda r, k: (r, 0)),
        out_shape=jax.ShapeDtypeStruct((NP, Co), jnp.float32),
    )(g4, feats_pad[:NP], wlo, whi, wc, bias.reshape(1, Co))


def kernel(feats, coords, weight, bias):
    coords = coords.astype(jnp.int32)
    lin = ((coords[:, 0] * D + coords[:, 1]) * H + coords[:, 2]) * W \
        + coords[:, 3]
    lin_pad = jnp.full((NP,), TAB, jnp.int32).at[:N].set(lin)
    keys = jnp.full((KEYR,), -1, jnp.int32).at[:N].set(lin)
    ids_pad = jnp.concatenate(
        [jnp.arange(N, dtype=jnp.int32),
         jnp.full((NP - N,), N, jnp.int32)])
    feats_pad = jnp.zeros((NPV, Ci), jnp.float32).at[:N].set(feats)
    fpk = jax.lax.bitcast_convert_type(
        feats_pad.astype(jnp.bfloat16).reshape(NPV, Ci // 2, 2), jnp.int32)
    wt = jnp.transpose(weight, (1, 2, 3, 4, 0)).reshape(27, Ci, Co)
    kidx = [(dz + 1) * 9 + (dy + 1) * 3 + (dx + 1) for dz, dy, dx in OFFS]
    wn = wt[jnp.array(kidx)]
    wbf = wn.astype(jnp.bfloat16).astype(jnp.float32)
    wlo = wbf[:, 0::2, :]
    whi = wbf[:, 1::2, :]
    wc = wt[13:14]

    table = _scatter_table(lin_pad, ids_pad)
    fidx = _lookup(lin_pad, keys, table)
    g = _featgather(fidx, fpk)
    out = _matmul(g, feats_pad, wlo, whi, wc, bias)
    return out[:N]


# speedup vs baseline: 12.7412x; 1.3662x over previous
import dataclasses

import jax
import jax.numpy as jnp
from jax import lax
from jax.experimental import pallas as pl
from jax.experimental.pallas import tpu as pltpu
from jax.experimental.pallas import tpu_sc as plsc

D = H = W = 128
B = 4
N = 100000
Ci = Co = 32
TAB = B * D * H * W
TABP = TAB + 8
NW = 32
CH = 128
NP = 102400
ZSP = 4096
NPV = NP + ZSP + 8
KEYR = N + 8
PPT = NP // NW
PC = PPT // CH
OFFS = [(dz, dy, dx)
        for dz in (-1, 0, 1) for dy in (-1, 0, 1) for dx in (-1, 0, 1)
        if (dz, dy, dx) != (0, 0, 0)]
KN = len(OFFS)
QC = KN * CH
RTOT = KN * NP
RW = RTOT // NW
CBJ = 25
NCB = RW // (CBJ * CH)
MBLK = 4096


def _sc_mesh():
    return plsc.VectorSubcoreMesh(core_axis_name="c", subcore_axis_name="s")


def _sc_params():
    cp = pltpu.CompilerParams()
    fields = pltpu.CompilerParams.__dataclass_fields__
    if "needs_layout_passes" in fields:
        cp = dataclasses.replace(cp, needs_layout_passes=False)
    if "use_tc_tiling_on_sc" in fields:
        cp = dataclasses.replace(cp, use_tc_tiling_on_sc=False)
    return cp


def _wid():
    return lax.axis_index("c") * 16 + lax.axis_index("s")


def _scatter_body(lin_hbm, ids_hbm, table_hbm, lin_v, ids_v, sem):
    wid = _wid()
    pltpu.sync_copy(lin_hbm.at[wid], lin_v)
    pltpu.sync_copy(ids_hbm.at[wid], ids_v)

    @pl.loop(0, PC)
    def _(c):
        pltpu.async_copy(ids_v.at[c], table_hbm.at[lin_v.at[c]], sem)

    @pl.loop(0, PC)
    def _(c):
        pltpu.make_async_copy(ids_v.at[c], table_hbm.at[lin_v.at[c]],
                              sem).wait()


def _scatter_table(lin_pad, ids_pad):
    k = pl.kernel(
        _scatter_body,
        out_type=jax.ShapeDtypeStruct((TABP,), jnp.int32),
        mesh=_sc_mesh(),
        scratch_types=[
            pltpu.VMEM((PC, CH), jnp.int32),
            pltpu.VMEM((PC, CH), jnp.int32),
            pltpu.SemaphoreType.DMA,
        ],
        compiler_params=_sc_params(),
    )
    return k(lin_pad.reshape(NW, PC, CH), ids_pad.reshape(NW, PC, CH))


def _mk_masks(l):
    x = l & 127
    y = (l >> 7) & 127
    z = (l >> 14) & 127
    vp = l < TAB
    return {
        (0, -1): (x > 0) & vp, (0, 1): (x < 127) & vp,
        (1, -1): (y > 0) & vp, (1, 1): (y < 127) & vp,
        (2, -1): (z > 0) & vp, (2, 1): (z < 127) & vp,
        None: vp,
    }


def _compute_q(lin_v, q_v, b, c):
    for s in range(CH // 16):
        l = lin_v[pl.ds(c * CH + s * 16, 16)]
        m = _mk_masks(l)
        for ki, (dz, dy, dx) in enumerate(OFFS):
            ok = m[None]
            for d, dd in ((2, dz), (1, dy), (0, dx)):
                if dd != 0:
                    ok = ok & m[(d, dd)]
            off = dz * (H * W) + dy * W + dx
            q_v[b, ki, pl.ds(s * 16, 16)] = jnp.where(ok, l + off, TAB)


def _lookup_body(lin_hbm, keys_hbm, table_hbm, fidx_hbm,
                 keys_v, lin_v, q_v, nid_v, f_v,
                 sem_g0, sem_g1, sem_w0, sem_w1):
    wid = _wid()
    pltpu.sync_copy(keys_hbm, keys_v)
    pltpu.sync_copy(lin_hbm.at[wid], lin_v)
    sem_gs = (sem_g0, sem_g1)

    def fire_gather(b):
        @pl.loop(0, KN)
        def _(ki):
            pltpu.async_copy(table_hbm.at[q_v.at[b, ki]], nid_v.at[b, ki],
                             sem_gs[b])

    def wait_gather(b):
        @pl.loop(0, KN)
        def _(ki):
            pltpu.make_async_copy(table_hbm.at[q_v.at[b, ki]],
                                  nid_v.at[b, ki], sem_gs[b]).wait()

    def verify_store(c, b, sem_w, drain):
        @pl.when(drain)
        def _():
            pltpu.make_async_copy(f_v.at[b], fidx_hbm.at[0], sem_w).wait()
        li = jax.lax.iota(jnp.int32, 16)
        for ki in range(KN):
            for s in range(CH // 16):
                sl = pl.ds(s * 16, 16)
                q16 = q_v[b, ki, sl]
                nc = jnp.minimum(jnp.maximum(nid_v[b, ki, sl], 0), KEYR - 1)
                key = plsc.load_gather(keys_v, [nc])
                dummy = NP + ((q16 + li * 257 + ki * 131 + s * 17)
                              & (ZSP - 1))
                f_v[b, ki, sl] = jnp.where(key == q16, nc, dummy)
        pltpu.async_copy(f_v.at[b], fidx_hbm.at[wid * PC + c], sem_w)

    _compute_q(lin_v, q_v, 0, 0)
    fire_gather(0)

    @pl.loop(0, PC // 2)
    def _(i):
        c0 = i * 2
        _compute_q(lin_v, q_v, 1, c0 + 1)
        fire_gather(1)
        wait_gather(0)
        verify_store(c0, 0, sem_w0, i > 0)

        @pl.when(c0 + 2 < PC)
        def _():
            _compute_q(lin_v, q_v, 0, c0 + 2)
            fire_gather(0)
        wait_gather(1)
        verify_store(c0 + 1, 1, sem_w1, i > 0)

    wait_gather(0)
    verify_store(PC - 1, 0, sem_w0, True)
    pltpu.make_async_copy(f_v.at[0], fidx_hbm.at[0], sem_w0).wait()
    pltpu.make_async_copy(f_v.at[1], fidx_hbm.at[0], sem_w1).wait()


def _lookup(lin_pad, keys, table):
    k = pl.kernel(
        _lookup_body,
        out_type=jax.ShapeDtypeStruct((NW * PC, KN, CH), jnp.int32),
        mesh=_sc_mesh(),
        scratch_types=[
            pltpu.VMEM((KEYR,), jnp.int32),
            pltpu.VMEM((PPT,), jnp.int32),
            pltpu.VMEM((2, KN, CH), jnp.int32),
            pltpu.VMEM((2, KN, CH), jnp.int32),
            pltpu.VMEM((2, KN, CH), jnp.int32),
            pltpu.SemaphoreType.DMA,
            pltpu.SemaphoreType.DMA,
            pltpu.SemaphoreType.DMA,
            pltpu.SemaphoreType.DMA,
        ],
        compiler_params=_sc_params(),
    )
    return k(lin_pad.reshape(NW, PPT), keys, table)


def _featgather_body(fidx_hbm, fpk_hbm, g_hbm,
                     idx_v, fbuf_v, sem_f0, sem_f1, sem_w0, sem_w1):
    wid = _wid()
    base = wid * (RW // CH)
    sem_fs = (sem_f0, sem_f1)
    sem_ws = (sem_w0, sem_w1)

    def load_idx(c, p):
        pltpu.sync_copy(fidx_hbm.at[pl.ds(base + c * CBJ, CBJ)],
                        idx_v.at[p])

    def fire_gather(p):
        @pl.loop(0, CBJ)
        def _(j):
            pltpu.async_copy(fpk_hbm.at[idx_v.at[p, j]], fbuf_v.at[p, j],
                             sem_fs[p])

    def wait_gather(p):
        @pl.loop(0, CBJ)
        def _(j):
            pltpu.make_async_copy(fpk_hbm.at[idx_v.at[p, j]],
                                  fbuf_v.at[p, j], sem_fs[p]).wait()

    def fire_write(c, p):
        pltpu.async_copy(fbuf_v.at[p],
                         g_hbm.at[pl.ds(base + c * CBJ, CBJ)], sem_ws[p])

    def wait_write(p):
        pltpu.make_async_copy(fbuf_v.at[p], g_hbm.at[pl.ds(base, CBJ)],
                              sem_ws[p]).wait()

    load_idx(0, 0)
    fire_gather(0)

    @pl.loop(0, NCB // 2)
    def _(i):
        c0 = i * 2

        @pl.when(i > 0)
        def _():
            wait_write(1)
        load_idx(c0 + 1, 1)
        fire_gather(1)
        wait_gather(0)
        fire_write(c0, 0)

        @pl.when(c0 + 2 < NCB)
        def _():
            wait_write(0)
            load_idx(c0 + 2, 0)
            fire_gather(0)
        wait_gather(1)
        fire_write(c0 + 1, 1)

    wait_write(0)
    wait_write(1)


def _featgather(fidx, fpk):
    k = pl.kernel(
        _featgather_body,
        out_type=jax.ShapeDtypeStruct((RTOT // CH, CH, Ci // 2), jnp.int32),
        mesh=_sc_mesh(),
        scratch_types=[
            pltpu.VMEM((2, CBJ, CH), jnp.int32),
            pltpu.VMEM((2, CBJ, CH, Ci // 2), jnp.int32),
            pltpu.SemaphoreType.DMA,
            pltpu.SemaphoreType.DMA,
            pltpu.SemaphoreType.DMA,
            pltpu.SemaphoreType.DMA,
        ],
        compiler_params=_sc_params(),
    )
    return k(fidx.reshape(RTOT // CH, CH), fpk)


def _mm_body(g_ref, f_ref, wlo_ref, whi_ref, wc_ref, b_ref, o_ref):
    k = pl.program_id(1)

    @pl.when(k == 0)
    def _():
        o_ref[...] = jnp.broadcast_to(b_ref[...], (MBLK, Co))

    @pl.when(k < KN)
    def _():
        gi = g_ref[...].reshape(MBLK, Ci // 2)
        lo = jax.lax.bitcast_convert_type(gi << 16, jnp.float32)
        hi = jax.lax.bitcast_convert_type(
            gi & jnp.int32(-65536), jnp.float32)
        o_ref[...] += (jnp.dot(lo, wlo_ref[0],
                               preferred_element_type=jnp.float32)
                       + jnp.dot(hi, whi_ref[0],
                                 preferred_element_type=jnp.float32))

    @pl.when(k == KN)
    def _():
        o_ref[...] += jnp.dot(f_ref[...], wc_ref[0],
                              preferred_element_type=jnp.float32)


def _matmul(g, feats_pad, wlo, whi, wc, bias):
    npc = NP // CH
    bpc = MBLK // CH
    g4 = g.reshape(npc, KN, CH, Ci // 2)
    grid = (npc // bpc, KN + 1)
    wsel = lambda r, k: (jnp.minimum(k, KN - 1), 0, 0)
    return pl.pallas_call(
        _mm_body,
        grid=grid,
        in_specs=[
            pl.BlockSpec((bpc, 1, CH, Ci // 2),
                         lambda r, k: (r, jnp.minimum(k, KN - 1), 0, 0)),
            pl.BlockSpec((MBLK, Ci), lambda r, k: (r, 0)),
            pl.BlockSpec((1, Ci // 2, Co), wsel),
            pl.BlockSpec((1, Ci // 2, Co), wsel),
            pl.BlockSpec((1, Ci, Co), lambda r, k: (0, 0, 0)),
            pl.BlockSpec((1, Co), lambda r, k: (0, 0)),
        ],
        out_specs=pl.BlockSpec((MBLK, Co), lambda r, k: (r, 0)),
        out_shape=jax.ShapeDtypeStruct((NP, Co), jnp.float32),
    )(g4, feats_pad[:NP], wlo, whi, wc, bias.reshape(1, Co))


def kernel(feats, coords, weight, bias):
    coords = coords.astype(jnp.int32)
    lin = ((coords[:, 0] * D + coords[:, 1]) * H + coords[:, 2]) * W \
        + coords[:, 3]
    lin_pad = jnp.full((NP,), TAB, jnp.int32).at[:N].set(lin)
    keys = jnp.full((KEYR,), -1, jnp.int32).at[:N].set(lin)
    ids_pad = jnp.concatenate(
        [jnp.arange(N, dtype=jnp.int32),
         jnp.full((NP - N,), N, jnp.int32)])
    feats_pad = jnp.zeros((NPV, Ci), jnp.float32).at[:N].set(feats)
    fpk = jax.lax.bitcast_convert_type(
        feats_pad.astype(jnp.bfloat16).reshape(NPV, Ci // 2, 2), jnp.int32)
    wt = jnp.transpose(weight, (1, 2, 3, 4, 0)).reshape(27, Ci, Co)
    kidx = [(dz + 1) * 9 + (dy + 1) * 3 + (dx + 1) for dz, dy, dx in OFFS]
    wn = wt[jnp.array(kidx)]
    wbf = wn.astype(jnp.bfloat16).astype(jnp.float32)
    wlo = wbf[:, 0::2, :]
    whi = wbf[:, 1::2, :]
    wc = wt[13:14]

    table = _scatter_table(lin_pad, ids_pad)
    fidx = _lookup(lin_pad, keys, table)
    g = _featgather(fidx, fpk)
    out = _matmul(g, feats_pad, wlo, whi, wc, bias)
    return out[:N]


# speedup vs baseline: 12.7459x; 1.0004x over previous
import dataclasses

import jax
import jax.numpy as jnp
from jax import lax
from jax.experimental import pallas as pl
from jax.experimental.pallas import tpu as pltpu
from jax.experimental.pallas import tpu_sc as plsc

D = H = W = 128
B = 4
N = 100000
Ci = Co = 32
TAB = B * D * H * W
TABP = TAB + 8
NW = 32
CH = 128
NP = 102400
ZSP = 4096
NPV = NP + ZSP + 8
KEYR = N + 8
PPT = NP // NW
PC = PPT // CH
OFFS = [(dz, dy, dx)
        for dz in (-1, 0, 1) for dy in (-1, 0, 1) for dx in (-1, 0, 1)
        if (dz, dy, dx) != (0, 0, 0)]
KN = len(OFFS)
QC = KN * CH
RTOT = KN * NP
RW = RTOT // NW
CBJ = 25
NCB = RW // (CBJ * CH)
MBLK = 4096


def _sc_mesh():
    return plsc.VectorSubcoreMesh(core_axis_name="c", subcore_axis_name="s")


def _sc_params():
    cp = pltpu.CompilerParams()
    fields = pltpu.CompilerParams.__dataclass_fields__
    if "needs_layout_passes" in fields:
        cp = dataclasses.replace(cp, needs_layout_passes=False)
    if "use_tc_tiling_on_sc" in fields:
        cp = dataclasses.replace(cp, use_tc_tiling_on_sc=False)
    return cp


def _wid():
    return lax.axis_index("c") * 16 + lax.axis_index("s")


def _scatter_body(lin_hbm, ids_hbm, table_hbm, lin_v, ids_v, sem):
    wid = _wid()
    pltpu.sync_copy(lin_hbm.at[wid], lin_v)
    pltpu.sync_copy(ids_hbm.at[wid], ids_v)

    @pl.loop(0, PC)
    def _(c):
        pltpu.async_copy(ids_v.at[c], table_hbm.at[lin_v.at[c]], sem)

    @pl.loop(0, PC)
    def _(c):
        pltpu.make_async_copy(ids_v.at[c], table_hbm.at[lin_v.at[c]],
                              sem).wait()


def _scatter_table(lin_pad, ids_pad):
    k = pl.kernel(
        _scatter_body,
        out_type=jax.ShapeDtypeStruct((TABP,), jnp.int32),
        mesh=_sc_mesh(),
        scratch_types=[
            pltpu.VMEM((PC, CH), jnp.int32),
            pltpu.VMEM((PC, CH), jnp.int32),
            pltpu.SemaphoreType.DMA,
        ],
        compiler_params=_sc_params(),
    )
    return k(lin_pad.reshape(NW, PC, CH), ids_pad.reshape(NW, PC, CH))


def _mk_masks(l):
    x = l & 127
    y = (l >> 7) & 127
    z = (l >> 14) & 127
    vp = l < TAB
    return {
        (0, -1): (x > 0) & vp, (0, 1): (x < 127) & vp,
        (1, -1): (y > 0) & vp, (1, 1): (y < 127) & vp,
        (2, -1): (z > 0) & vp, (2, 1): (z < 127) & vp,
        None: vp,
    }


def _compute_q(lin_v, q_v, b, c):
    for s in range(CH // 16):
        l = lin_v[pl.ds(c * CH + s * 16, 16)]
        m = _mk_masks(l)
        for ki, (dz, dy, dx) in enumerate(OFFS):
            ok = m[None]
            for d, dd in ((2, dz), (1, dy), (0, dx)):
                if dd != 0:
                    ok = ok & m[(d, dd)]
            off = dz * (H * W) + dy * W + dx
            q_v[b, pl.ds(ki * CH + s * 16, 16)] = jnp.where(ok, l + off, TAB)


def _lookup_body(lin_hbm, keys_hbm, table_hbm, fidx_hbm,
                 keys_v, lin_v, q_v, nid_v, f_v,
                 sem_g0, sem_g1, sem_w0, sem_w1):
    wid = _wid()
    pltpu.sync_copy(keys_hbm, keys_v)
    pltpu.sync_copy(lin_hbm.at[wid], lin_v)
    sem_gs = (sem_g0, sem_g1)

    def fire_gather(b):
        pltpu.async_copy(table_hbm.at[q_v.at[b]], nid_v.at[b], sem_gs[b])

    def wait_gather(b):
        pltpu.make_async_copy(table_hbm.at[q_v.at[b]], nid_v.at[b],
                              sem_gs[b]).wait()

    def verify_store(c, b, sem_w, drain):
        @pl.when(drain)
        def _():
            pltpu.make_async_copy(f_v.at[b], fidx_hbm.at[0], sem_w).wait()
        li = jax.lax.iota(jnp.int32, 16)
        for ki in range(KN):
            for s in range(CH // 16):
                sl = pl.ds(ki * CH + s * 16, 16)
                q16 = q_v[b, sl]
                nc = jnp.minimum(jnp.maximum(nid_v[b, sl], 0), KEYR - 1)
                key = plsc.load_gather(keys_v, [nc])
                dummy = NP + ((q16 + li * 257 + ki * 131 + s * 17)
                              & (ZSP - 1))
                f_v[b, sl] = jnp.where(key == q16, nc, dummy)
        pltpu.async_copy(f_v.at[b], fidx_hbm.at[wid * PC + c], sem_w)

    _compute_q(lin_v, q_v, 0, 0)
    fire_gather(0)

    @pl.loop(0, PC // 2)
    def _(i):
        c0 = i * 2
        _compute_q(lin_v, q_v, 1, c0 + 1)
        fire_gather(1)
        wait_gather(0)
        verify_store(c0, 0, sem_w0, i > 0)

        @pl.when(c0 + 2 < PC)
        def _():
            _compute_q(lin_v, q_v, 0, c0 + 2)
            fire_gather(0)
        wait_gather(1)
        verify_store(c0 + 1, 1, sem_w1, i > 0)

    wait_gather(0)
    verify_store(PC - 1, 0, sem_w0, True)
    pltpu.make_async_copy(f_v.at[0], fidx_hbm.at[0], sem_w0).wait()
    pltpu.make_async_copy(f_v.at[1], fidx_hbm.at[0], sem_w1).wait()


def _lookup(lin_pad, keys, table):
    k = pl.kernel(
        _lookup_body,
        out_type=jax.ShapeDtypeStruct((NW * PC, QC), jnp.int32),
        mesh=_sc_mesh(),
        scratch_types=[
            pltpu.VMEM((KEYR,), jnp.int32),
            pltpu.VMEM((PPT,), jnp.int32),
            pltpu.VMEM((2, QC), jnp.int32),
            pltpu.VMEM((2, QC), jnp.int32),
            pltpu.VMEM((2, QC), jnp.int32),
            pltpu.SemaphoreType.DMA,
            pltpu.SemaphoreType.DMA,
            pltpu.SemaphoreType.DMA,
            pltpu.SemaphoreType.DMA,
        ],
        compiler_params=_sc_params(),
    )
    return k(lin_pad.reshape(NW, PPT), keys, table)


def _featgather_body(fidx_hbm, fpk_hbm, g_hbm,
                     idx_v, fbuf_v, sem_f0, sem_f1, sem_w0, sem_w1):
    wid = _wid()
    base = wid * RW
    slot = CBJ * CH
    sem_fs = (sem_f0, sem_f1)
    sem_ws = (sem_w0, sem_w1)

    def load_idx(c, p):
        pltpu.sync_copy(fidx_hbm.at[pl.ds(base + c * slot, slot)],
                        idx_v.at[p])

    def fire_gather(p):
        pltpu.async_copy(fpk_hbm.at[idx_v.at[p]], fbuf_v.at[p], sem_fs[p])

    def wait_gather(p):
        pltpu.make_async_copy(fpk_hbm.at[idx_v.at[p]], fbuf_v.at[p],
                              sem_fs[p]).wait()

    def fire_write(c, p):
        pltpu.async_copy(fbuf_v.at[p],
                         g_hbm.at[pl.ds(base + c * slot, slot)], sem_ws[p])

    def wait_write(p):
        pltpu.make_async_copy(fbuf_v.at[p], g_hbm.at[pl.ds(base, slot)],
                              sem_ws[p]).wait()

    load_idx(0, 0)
    fire_gather(0)

    @pl.loop(0, NCB // 2)
    def _(i):
        c0 = i * 2

        @pl.when(i > 0)
        def _():
            wait_write(1)
        load_idx(c0 + 1, 1)
        fire_gather(1)
        wait_gather(0)
        fire_write(c0, 0)

        @pl.when(c0 + 2 < NCB)
        def _():
            wait_write(0)
            load_idx(c0 + 2, 0)
            fire_gather(0)
        wait_gather(1)
        fire_write(c0 + 1, 1)

    wait_write(0)
    wait_write(1)


def _featgather(fidx, fpk):
    k = pl.kernel(
        _featgather_body,
        out_type=jax.ShapeDtypeStruct((RTOT, Ci // 2), jnp.int32),
        mesh=_sc_mesh(),
        scratch_types=[
            pltpu.VMEM((2, CBJ * CH), jnp.int32),
            pltpu.VMEM((2, CBJ * CH, Ci // 2), jnp.int32),
            pltpu.SemaphoreType.DMA,
            pltpu.SemaphoreType.DMA,
            pltpu.SemaphoreType.DMA,
            pltpu.SemaphoreType.DMA,
        ],
        compiler_params=_sc_params(),
    )
    return k(fidx.reshape(RTOT), fpk)


def _mm_body(g_ref, f_ref, wlo_ref, whi_ref, wc_ref, b_ref, o_ref):
    k = pl.program_id(1)

    @pl.when(k == 0)
    def _():
        o_ref[...] = jnp.broadcast_to(b_ref[...], (MBLK, Co))

    @pl.when(k < KN)
    def _():
        gi = g_ref[...].reshape(MBLK, Ci // 2)
        lo = jax.lax.bitcast_convert_type(gi << 16, jnp.float32)
        hi = jax.lax.bitcast_convert_type(
            gi & jnp.int32(-65536), jnp.float32)
        o_ref[...] += (jnp.dot(lo, wlo_ref[0],
                               preferred_element_type=jnp.float32)
                       + jnp.dot(hi, whi_ref[0],
                                 preferred_element_type=jnp.float32))

    @pl.when(k == KN)
    def _():
        o_ref[...] += jnp.dot(f_ref[...], wc_ref[0],
                              preferred_element_type=jnp.float32)


def _matmul(g, feats_pad, wlo, whi, wc, bias):
    npc = NP // CH
    bpc = MBLK // CH
    g4 = g.reshape(npc, KN, CH, Ci // 2)
    grid = (npc // bpc, KN + 1)
    wsel = lambda r, k: (jnp.minimum(k, KN - 1), 0, 0)
    return pl.pallas_call(
        _mm_body,
        grid=grid,
        in_specs=[
            pl.BlockSpec((bpc, 1, CH, Ci // 2),
                         lambda r, k: (r, jnp.minimum(k, KN - 1), 0, 0)),
            pl.BlockSpec((MBLK, Ci), lambda r, k: (r, 0)),
            pl.BlockSpec((1, Ci // 2, Co), wsel),
            pl.BlockSpec((1, Ci // 2, Co), wsel),
            pl.BlockSpec((1, Ci, Co), lambda r, k: (0, 0, 0)),
            pl.BlockSpec((1, Co), lambda r, k: (0, 0)),
        ],
        out_specs=pl.BlockSpec((MBLK, Co), lambda r, k: (r, 0)),
        out_shape=jax.ShapeDtypeStruct((NP, Co), jnp.float32),
    )(g4, feats_pad[:NP], wlo, whi, wc, bias.reshape(1, Co))


def kernel(feats, coords, weight, bias):
    coords = coords.astype(jnp.int32)
    lin = ((coords[:, 0] * D + coords[:, 1]) * H + coords[:, 2]) * W \
        + coords[:, 3]
    lin_pad = jnp.full((NP,), TAB, jnp.int32).at[:N].set(lin)
    keys = jnp.full((KEYR,), -1, jnp.int32).at[:N].set(lin)
    ids_pad = jnp.concatenate(
        [jnp.arange(N, dtype=jnp.int32),
         jnp.full((NP - N,), N, jnp.int32)])
    feats_pad = jnp.zeros((NPV, Ci), jnp.float32).at[:N].set(feats)
    fpk = jax.lax.bitcast_convert_type(
        feats_pad.astype(jnp.bfloat16).reshape(NPV, Ci // 2, 2), jnp.int32)
    wt = jnp.transpose(weight, (1, 2, 3, 4, 0)).reshape(27, Ci, Co)
    kidx = [(dz + 1) * 9 + (dy + 1) * 3 + (dx + 1) for dz, dy, dx in OFFS]
    wn = wt[jnp.array(kidx)]
    wbf = wn.astype(jnp.bfloat16).astype(jnp.float32)
    wlo = wbf[:, 0::2, :]
    whi = wbf[:, 1::2, :]
    wc = wt[13:14]

    table = _scatter_table(lin_pad, ids_pad)
    fidx = _lookup(lin_pad, keys, table)
    g = _featgather(fidx, fpk)
    out = _matmul(g, feats_pad, wlo, whi, wc, bias)
    return out[:N]


# speedup vs baseline: 20.3332x; 1.5953x over previous
import dataclasses

import jax
import jax.numpy as jnp
from jax import lax
from jax.experimental import pallas as pl
from jax.experimental.pallas import tpu as pltpu
from jax.experimental.pallas import tpu_sc as plsc

D = H = W = 128
B = 4
N = 100000
Ci = Co = 32
TAB = B * D * H * W
TABP = TAB + 8
NW = 32
CH = 128
NP = 102400
ZSP = 4096
NPV = NP + ZSP + 8
KEYR = N + 8
PPT = NP // NW
PC = PPT // CH
OFFS = [(dz, dy, dx)
        for dz in (-1, 0, 1) for dy in (-1, 0, 1) for dx in (-1, 0, 1)
        if (dz, dy, dx) != (0, 0, 0)]
KN = len(OFFS)
QC = KN * CH
RTOT = KN * NP
RW = RTOT // NW
CBJ = 25
NCB = RW // (CBJ * CH)
MBLK = 4096


def _sc_mesh():
    return plsc.VectorSubcoreMesh(core_axis_name="c", subcore_axis_name="s")


def _sc_params():
    cp = pltpu.CompilerParams()
    fields = pltpu.CompilerParams.__dataclass_fields__
    if "needs_layout_passes" in fields:
        cp = dataclasses.replace(cp, needs_layout_passes=False)
    if "use_tc_tiling_on_sc" in fields:
        cp = dataclasses.replace(cp, use_tc_tiling_on_sc=False)
    return cp


def _wid():
    return lax.axis_index("c") * 16 + lax.axis_index("s")


def _scatter_body(lin_hbm, ids_hbm, table_hbm, lin_v, ids_v, sem):
    wid = _wid()
    pltpu.sync_copy(lin_hbm.at[wid], lin_v)
    pltpu.sync_copy(ids_hbm.at[wid], ids_v)

    @pl.loop(0, PC)
    def _(c):
        pltpu.async_copy(ids_v.at[c], table_hbm.at[lin_v.at[c]], sem)

    @pl.loop(0, PC)
    def _(c):
        pltpu.make_async_copy(ids_v.at[c], table_hbm.at[lin_v.at[c]],
                              sem).wait()


def _scatter_table(lin_pad, ids_pad):
    k = pl.kernel(
        _scatter_body,
        out_type=jax.ShapeDtypeStruct((TABP,), jnp.int32),
        mesh=_sc_mesh(),
        scratch_types=[
            pltpu.VMEM((PC, CH), jnp.int32),
            pltpu.VMEM((PC, CH), jnp.int32),
            pltpu.SemaphoreType.DMA,
        ],
        compiler_params=_sc_params(),
    )
    return k(lin_pad.reshape(NW, PC, CH), ids_pad.reshape(NW, PC, CH))


def _mk_masks(l):
    x = l & 127
    y = (l >> 7) & 127
    z = (l >> 14) & 127
    vp = l < TAB
    return {
        (0, -1): (x > 0) & vp, (0, 1): (x < 127) & vp,
        (1, -1): (y > 0) & vp, (1, 1): (y < 127) & vp,
        (2, -1): (z > 0) & vp, (2, 1): (z < 127) & vp,
        None: vp,
    }


def _compute_q(lin_v, q_v, b, c):
    for s in range(CH // 16):
        l = lin_v[pl.ds(c * CH + s * 16, 16)]
        m = _mk_masks(l)
        for ki, (dz, dy, dx) in enumerate(OFFS):
            ok = m[None]
            for d, dd in ((2, dz), (1, dy), (0, dx)):
                if dd != 0:
                    ok = ok & m[(d, dd)]
            off = dz * (H * W) + dy * W + dx
            q_v[b, pl.ds(ki * CH + s * 16, 16)] = jnp.where(ok, l + off, TAB)


def _lookup_body(lin_hbm, keys_hbm, table_hbm, fidx_hbm,
                 keys_v, lin_v, q_v, nid_v, f_v,
                 sem_g0, sem_g1, sem_w0, sem_w1):
    wid = _wid()
    pltpu.sync_copy(keys_hbm, keys_v)
    pltpu.sync_copy(lin_hbm.at[wid], lin_v)
    sem_gs = (sem_g0, sem_g1)

    def fire_gather(b):
        pltpu.async_copy(table_hbm.at[q_v.at[b]], nid_v.at[b], sem_gs[b])

    def wait_gather(b):
        pltpu.make_async_copy(table_hbm.at[q_v.at[b]], nid_v.at[b],
                              sem_gs[b]).wait()

    def verify_store(c, b, sem_w, drain):
        @pl.when(drain)
        def _():
            pltpu.make_async_copy(f_v.at[b], fidx_hbm.at[0], sem_w).wait()
        li = jax.lax.iota(jnp.int32, 16)
        for ki in range(KN):
            for s in range(CH // 16):
                sl = pl.ds(ki * CH + s * 16, 16)
                q16 = q_v[b, sl]
                nc = jnp.minimum(jnp.maximum(nid_v[b, sl], 0), KEYR - 1)
                key = plsc.load_gather(keys_v, [nc])
                dummy = NP + ((q16 + li * 257 + ki * 131 + s * 17)
                              & (ZSP - 1))
                f_v[b, sl] = jnp.where(key == q16, nc, dummy)
        pltpu.async_copy(f_v.at[b], fidx_hbm.at[wid * PC + c], sem_w)

    _compute_q(lin_v, q_v, 0, 0)
    fire_gather(0)

    @pl.loop(0, PC // 2)
    def _(i):
        c0 = i * 2
        _compute_q(lin_v, q_v, 1, c0 + 1)
        fire_gather(1)
        wait_gather(0)
        verify_store(c0, 0, sem_w0, i > 0)

        @pl.when(c0 + 2 < PC)
        def _():
            _compute_q(lin_v, q_v, 0, c0 + 2)
            fire_gather(0)
        wait_gather(1)
        verify_store(c0 + 1, 1, sem_w1, i > 0)

    wait_gather(0)
    verify_store(PC - 1, 0, sem_w0, True)
    pltpu.make_async_copy(f_v.at[0], fidx_hbm.at[0], sem_w0).wait()
    pltpu.make_async_copy(f_v.at[1], fidx_hbm.at[0], sem_w1).wait()


def _lookup(lin_pad, keys, table):
    k = pl.kernel(
        _lookup_body,
        out_type=jax.ShapeDtypeStruct((NW * PC, QC), jnp.int32),
        mesh=_sc_mesh(),
        scratch_types=[
            pltpu.VMEM((KEYR,), jnp.int32),
            pltpu.VMEM((PPT,), jnp.int32),
            pltpu.VMEM((2, QC), jnp.int32),
            pltpu.VMEM((2, QC), jnp.int32),
            pltpu.VMEM((2, QC), jnp.int32),
            pltpu.SemaphoreType.DMA,
            pltpu.SemaphoreType.DMA,
            pltpu.SemaphoreType.DMA,
            pltpu.SemaphoreType.DMA,
        ],
        compiler_params=_sc_params(),
    )
    return k(lin_pad.reshape(NW, PPT), keys, table)


def _featgather_body(fidx_hbm, fpk_hbm, g_hbm,
                     idx_v, fbuf_v, sem_f0, sem_f1, sem_w0, sem_w1):
    wid = _wid()
    base = wid * RW
    slot = CBJ * CH
    sem_fs = (sem_f0, sem_f1)
    sem_ws = (sem_w0, sem_w1)

    def load_idx(c, p):
        pltpu.sync_copy(fidx_hbm.at[pl.ds(base + c * slot, slot)],
                        idx_v.at[p])

    def fire_gather(p):
        pltpu.async_copy(fpk_hbm.at[idx_v.at[p]], fbuf_v.at[p], sem_fs[p])

    def wait_gather(p):
        pltpu.make_async_copy(fpk_hbm.at[idx_v.at[p]], fbuf_v.at[p],
                              sem_fs[p]).wait()

    def fire_write(c, p):
        pltpu.async_copy(fbuf_v.at[p],
                         g_hbm.at[pl.ds(base + c * slot, slot)], sem_ws[p])

    def wait_write(p):
        pltpu.make_async_copy(fbuf_v.at[p], g_hbm.at[pl.ds(base, slot)],
                              sem_ws[p]).wait()

    load_idx(0, 0)
    fire_gather(0)

    @pl.loop(0, NCB // 2)
    def _(i):
        c0 = i * 2

        @pl.when(i > 0)
        def _():
            wait_write(1)
        load_idx(c0 + 1, 1)
        fire_gather(1)
        wait_gather(0)
        fire_write(c0, 0)

        @pl.when(c0 + 2 < NCB)
        def _():
            wait_write(0)
            load_idx(c0 + 2, 0)
            fire_gather(0)
        wait_gather(1)
        fire_write(c0 + 1, 1)

    wait_write(0)
    wait_write(1)


def _featgather(fidx, fpk):
    k = pl.kernel(
        _featgather_body,
        out_type=jax.ShapeDtypeStruct((RTOT, Ci // 2), jnp.int32),
        mesh=_sc_mesh(),
        scratch_types=[
            pltpu.VMEM((2, CBJ * CH), jnp.int32),
            pltpu.VMEM((2, CBJ * CH, Ci // 2), jnp.int32),
            pltpu.SemaphoreType.DMA,
            pltpu.SemaphoreType.DMA,
            pltpu.SemaphoreType.DMA,
            pltpu.SemaphoreType.DMA,
        ],
        compiler_params=_sc_params(),
    )
    return k(fidx.reshape(RTOT), fpk)


NROW = MBLK // 8


def _mm_body(g_ref, f_ref, wlo_ref, whi_ref, wc_ref, b_ref, o_ref):
    k = pl.program_id(1)

    @pl.when(k == 0)
    def _():
        o_ref[...] = jnp.broadcast_to(b_ref[...], (NROW, 8 * Co))

    @pl.when(k < KN)
    def _():
        gi = g_ref[...].reshape(NROW // 16, 16, 128).reshape(NROW, 128)
        lo = jax.lax.bitcast_convert_type(gi << 16, jnp.float32)
        hi = jax.lax.bitcast_convert_type(
            gi & jnp.int32(-65536), jnp.float32)
        o_ref[...] += (jnp.dot(lo, wlo_ref[0],
                               preferred_element_type=jnp.float32)
                       + jnp.dot(hi, whi_ref[0],
                                 preferred_element_type=jnp.float32))

    @pl.when(k == KN)
    def _():
        o_ref[...] += jnp.dot(f_ref[...], wc_ref[...],
                              preferred_element_type=jnp.float32)


def _matmul(g, feats_pad, wlo8, whi8, wc8, bias8):
    npc = NP // CH
    bpc = MBLK // CH
    g4 = g.reshape(npc, KN, CH * (Ci // 2) // 128, 128)
    grid = (npc // bpc, KN + 1)
    wsel = lambda r, k: (jnp.minimum(k, KN - 1), 0, 0)
    out = pl.pallas_call(
        _mm_body,
        grid=grid,
        in_specs=[
            pl.BlockSpec((bpc, 1, CH * (Ci // 2) // 128, 128),
                         lambda r, k: (r, jnp.minimum(k, KN - 1), 0, 0)),
            pl.BlockSpec((NROW, 8 * Ci), lambda r, k: (r, 0)),
            pl.BlockSpec((1, 128, 8 * Co), wsel),
            pl.BlockSpec((1, 128, 8 * Co), wsel),
            pl.BlockSpec((8 * Ci, 8 * Co), lambda r, k: (0, 0)),
            pl.BlockSpec((1, 8 * Co), lambda r, k: (0, 0)),
        ],
        out_specs=pl.BlockSpec((NROW, 8 * Co), lambda r, k: (r, 0)),
        out_shape=jax.ShapeDtypeStruct((NP // 8, 8 * Co), jnp.float32),
    )(g4, feats_pad[:NP].reshape(NP // 8, 8 * Ci), wlo8, whi8, wc8, bias8)
    return out.reshape(NP, Co)


def kernel(feats, coords, weight, bias):
    coords = coords.astype(jnp.int32)
    lin = ((coords[:, 0] * D + coords[:, 1]) * H + coords[:, 2]) * W \
        + coords[:, 3]
    lin_pad = jnp.full((NP,), TAB, jnp.int32).at[:N].set(lin)
    keys = jnp.full((KEYR,), -1, jnp.int32).at[:N].set(lin)
    ids_pad = jnp.concatenate(
        [jnp.arange(N, dtype=jnp.int32),
         jnp.full((NP - N,), N, jnp.int32)])
    feats_pad = jnp.zeros((NPV, Ci), jnp.float32).at[:N].set(feats)
    fpk = jax.lax.bitcast_convert_type(
        feats_pad.astype(jnp.bfloat16).reshape(NPV, Ci // 2, 2), jnp.int32)
    wt = jnp.transpose(weight, (1, 2, 3, 4, 0)).reshape(27, Ci, Co)
    kidx = [(dz + 1) * 9 + (dy + 1) * 3 + (dx + 1) for dz, dy, dx in OFFS]
    wn = wt[jnp.array(kidx)]
    wbf = wn.astype(jnp.bfloat16).astype(jnp.float32)
    eye8 = jnp.eye(8, dtype=jnp.float32)
    wlo8 = jnp.stack([jnp.kron(eye8, wbf[i, 0::2, :]) for i in range(KN)])
    whi8 = jnp.stack([jnp.kron(eye8, wbf[i, 1::2, :]) for i in range(KN)])
    wc8 = jnp.kron(eye8, wt[13])
    bias8 = jnp.tile(bias, 8).reshape(1, 8 * Co)

    table = _scatter_table(lin_pad, ids_pad)
    fidx = _lookup(lin_pad, keys, table)
    g = _featgather(fidx, fpk)
    out = _matmul(g, feats_pad, wlo8, whi8, wc8, bias8)
    return out[:N]
